# baseline (speedup 1.0000x reference)
import functools

import jax
import jax.numpy as jnp
from jax import lax
from jax.experimental import pallas as pl
from jax.experimental.pallas import tpu as pltpu

_EPS = 1e-5
_SLOPE = 0.1
_KP_EXTENT = 1.2
_H = 16
_KP = 15
_KPAD = 16
_CM = 64


def _lrelu(y):
    return jnp.where(y >= 0.0, y, _SLOPE * y)


def _kpconv_kernel(idx_ref, src_ref, qaug_ref, fo_ref, wo_ref, iv_ref,
                   slab_ref, ism_ref, sem, *, tn):
    cp = pltpu.make_async_copy(idx_ref.at[0, 0], ism_ref, sem)
    cp.start()
    cp.wait()

    def body(ni, carry):
        for h in range(_H):
            j = ism_ref[ni * _H + h]
            slab_ref[h * tn + ni] = src_ref[j]
        return carry

    lax.fori_loop(0, tn, body, 0)

    sub = 64
    for s in range(tn // sub):
        base = s * sub
        t2 = qaug_ref[pl.ds(base, sub), :, 0:_KPAD]
        qx = qaug_ref[pl.ds(base, sub), :, 16:17]
        qy = qaug_ref[pl.ds(base, sub), :, 17:18]
        qz = qaug_ref[pl.ds(base, sub), :, 18:19]
        cnt = jnp.zeros((sub, 1, 1), jnp.float32)
        for h in range(_H):
            sl = slab_ref[pl.ds(h * tn + base, sub)]
            skp16 = sl[:, :, 64:80]
            s2v = sl[:, :, 79:80]
            sdq = (sl[:, :, 80:81] * qx + sl[:, :, 81:82] * qy
                   + sl[:, :, 82:83] * qz)
            sqd = jnp.maximum(s2v - 2.0 * sdq + (t2 - 2.0 * skp16), 0.0)
            w_h = jnp.maximum(1.0 - jnp.sqrt(sqd) * (1.0 / _KP_EXTENT), 0.0)
            wo_ref[pl.ds(base, sub), :, h * _KPAD:(h + 1) * _KPAD] = w_h
            fo_ref[base * _H + h:(base + sub) * _H + h:_H] = sl[:, :, 0:_CM]
            rs = jnp.sum(sl[:, :, 0:_CM], axis=2, keepdims=True)
            cnt = cnt + (rs > 0.0).astype(jnp.float32)
        iv_ref[pl.ds(base, sub), :, 0:1] = 1.0 / jnp.maximum(cnt, 1.0)


def _kpconv(src, qaug, idx3, n, tn):
    t = n // tn
    return pl.pallas_call(
        functools.partial(_kpconv_kernel, tn=tn),
        out_shape=(jax.ShapeDtypeStruct((n * _H, 1, _CM), jnp.float32),
                   jax.ShapeDtypeStruct((n, 1, _KPAD * _H), jnp.float32),
                   jax.ShapeDtypeStruct((n, 1, 8), jnp.float32)),
        grid=(t,),
        in_specs=[pl.BlockSpec((1, 1, tn * _H), lambda i: (i, 0, 0)),
                  pl.BlockSpec(src.shape, lambda i: (0, 0, 0)),
                  pl.BlockSpec((tn, 1, 24), lambda i: (i, 0, 0))],
        out_specs=(pl.BlockSpec((tn * _H, 1, _CM), lambda i: (i, 0, 0)),
                   pl.BlockSpec((tn, 1, _KPAD * _H), lambda i: (i, 0, 0)),
                   pl.BlockSpec((tn, 1, 8), lambda i: (i, 0, 0))),
        scratch_shapes=[pltpu.VMEM((tn * _H, 1, 128), jnp.float32),
                        pltpu.SMEM((tn * _H,), jnp.int32),
                        pltpu.SemaphoreType.DMA],
        compiler_params=pltpu.CompilerParams(
            dimension_semantics=("parallel",),
            disable_bounds_checks=True,
            vmem_limit_bytes=60 * 1024 * 1024),
    )(idx3, src, qaug)


def _gemm_stats_kernel(x_ref, w_ref, ps_ref, pq_ref):
    y = jnp.dot(x_ref[...], w_ref[...], preferred_element_type=jnp.float32)
    ps_ref[0, 0, :] = jnp.sum(y, axis=0)
    pq_ref[0, 0, :] = jnp.sum(y * y, axis=0)


def _gemm_stats(x, w, tn):
    n, kdim = x.shape
    c = w.shape[1]
    t = n // tn
    return pl.pallas_call(
        _gemm_stats_kernel,
        out_shape=(jax.ShapeDtypeStruct((t, 1, c), jnp.float32),
                   jax.ShapeDtypeStruct((t, 1, c), jnp.float32)),
        grid=(t,),
        in_specs=[pl.BlockSpec((tn, kdim), lambda i: (i, 0)),
                  pl.BlockSpec((kdim, c), lambda i: (0, 0))],
        out_specs=(pl.BlockSpec((1, 1, c), lambda i: (i, 0, 0)),
                   pl.BlockSpec((1, 1, c), lambda i: (i, 0, 0))),
        compiler_params=pltpu.CompilerParams(
            dimension_semantics=("parallel",)),
    )(x, w)


def _scale_shift(ps, pq, n):
    s = jnp.sum(ps[:, 0, :], axis=0)
    q = jnp.sum(pq[:, 0, :], axis=0)
    mean = s / n
    var = q / n - mean * mean
    inv = lax.rsqrt(var + _EPS)
    return inv[None, :], (-mean * inv)[None, :]


def _gemm_affine_kernel(x_ref, w_ref, a_ref, b_ref, o_ref):
    y = jnp.dot(x_ref[...], w_ref[...], preferred_element_type=jnp.float32)
    o_ref[...] = _lrelu(y * a_ref[...] + b_ref[...])


def _gemm_affine(x, w, a, b, tn):
    n, kdim = x.shape
    c = w.shape[1]
    t = n // tn
    return pl.pallas_call(
        _gemm_affine_kernel,
        out_shape=jax.ShapeDtypeStruct((n, c), jnp.float32),
        grid=(t,),
        in_specs=[pl.BlockSpec((tn, kdim), lambda i: (i, 0)),
                  pl.BlockSpec((kdim, c), lambda i: (0, 0)),
                  pl.BlockSpec((1, c), lambda i: (0, 0)),
                  pl.BlockSpec((1, c), lambda i: (0, 0))],
        out_specs=pl.BlockSpec((tn, c), lambda i: (i, 0)),
        compiler_params=pltpu.CompilerParams(
            dimension_semantics=("parallel",)),
    )(x, w, a, b)


def _conv_gemm_kernel(wt_ref, w_ref, inv_ref, o_ref, ps_ref, pq_ref):
    y = jnp.dot(wt_ref[...], w_ref[...], preferred_element_type=jnp.float32)
    y = y * inv_ref[...]
    o_ref[...] = y
    ps_ref[0, 0, :] = jnp.sum(y, axis=0)
    pq_ref[0, 0, :] = jnp.sum(y * y, axis=0)


def _conv_gemm(weighted, w, inv, tn):
    n, kdim = weighted.shape
    c = w.shape[1]
    t = n // tn
    return pl.pallas_call(
        _conv_gemm_kernel,
        out_shape=(jax.ShapeDtypeStruct((n, c), jnp.float32),
                   jax.ShapeDtypeStruct((t, 1, c), jnp.float32),
                   jax.ShapeDtypeStruct((t, 1, c), jnp.float32)),
        grid=(t,),
        in_specs=[pl.BlockSpec((tn, kdim), lambda i: (i, 0)),
                  pl.BlockSpec((kdim, c), lambda i: (0, 0)),
                  pl.BlockSpec((tn, 1), lambda i: (i, 0))],
        out_specs=(pl.BlockSpec((tn, c), lambda i: (i, 0)),
                   pl.BlockSpec((1, 1, c), lambda i: (i, 0, 0)),
                   pl.BlockSpec((1, 1, c), lambda i: (i, 0, 0))),
        compiler_params=pltpu.CompilerParams(
            dimension_semantics=("parallel",)),
    )(weighted, w, inv)


def _final_stats_kernel(cv_ref, ft_ref, a3_ref, b3_ref, w2_ref, ws_ref,
                        ps2_ref, pq2_ref, pss_ref, pqs_ref):
    x3 = _lrelu(cv_ref[...] * a3_ref[...] + b3_ref[...])
    y2 = jnp.dot(x3, w2_ref[...], preferred_element_type=jnp.float32)
    sc = jnp.dot(ft_ref[...], ws_ref[...], preferred_element_type=jnp.float32)
    ps2_ref[0, 0, :] = jnp.sum(y2, axis=0)
    pq2_ref[0, 0, :] = jnp.sum(y2 * y2, axis=0)
    pss_ref[0, 0, :] = jnp.sum(sc, axis=0)
    pqs_ref[0, 0, :] = jnp.sum(sc * sc, axis=0)


def _final_stats(conv, feat, a3, b3, w2, ws, tn):
    n, cm = conv.shape
    cin = feat.shape[1]
    c = w2.shape[1]
    t = n // tn
    stat = jax.ShapeDtypeStruct((t, 1, c), jnp.float32)
    return pl.pallas_call(
        _final_stats_kernel,
        out_shape=(stat, stat, stat, stat),
        grid=(t,),
        in_specs=[pl.BlockSpec((tn, cm), lambda i: (i, 0)),
                  pl.BlockSpec((tn, cin), lambda i: (i, 0)),
                  pl.BlockSpec((1, cm), lambda i: (0, 0)),
                  pl.BlockSpec((1, cm), lambda i: (0, 0)),
                  pl.BlockSpec((cm, c), lambda i: (0, 0)),
                  pl.BlockSpec((cin, c), lambda i: (0, 0))],
        out_specs=(pl.BlockSpec((1, 1, c), lambda i: (i, 0, 0)),) * 4,
        compiler_params=pltpu.CompilerParams(
            dimension_semantics=("parallel",)),
    )(conv, feat, a3, b3, w2, ws)


def _final_apply_kernel(cv_ref, ft_ref, a3_ref, b3_ref, w2_ref, ws_ref,
                        a2_ref, b2_ref, as_ref, bs_ref, o_ref):
    x3 = _lrelu(cv_ref[...] * a3_ref[...] + b3_ref[...])
    y2 = jnp.dot(x3, w2_ref[...], preferred_element_type=jnp.float32)
    sc = jnp.dot(ft_ref[...], ws_ref[...], preferred_element_type=jnp.float32)
    y = y2 * a2_ref[...] + b2_ref[...] + (sc * as_ref[...] + bs_ref[...])
    o_ref[...] = _lrelu(y)


def _final_apply(conv, feat, a3, b3, w2, ws, a2, b2, a_s, b_s, tn):
    n, cm = conv.shape
    cin = feat.shape[1]
    c = w2.shape[1]
    t = n // tn
    vec_c = pl.BlockSpec((1, c), lambda i: (0, 0))
    return pl.pallas_call(
        _final_apply_kernel,
        out_shape=jax.ShapeDtypeStruct((n, c), jnp.float32),
        grid=(t,),
        in_specs=[pl.BlockSpec((tn, cm), lambda i: (i, 0)),
                  pl.BlockSpec((tn, cin), lambda i: (i, 0)),
                  pl.BlockSpec((1, cm), lambda i: (0, 0)),
                  pl.BlockSpec((1, cm), lambda i: (0, 0)),
                  pl.BlockSpec((cm, c), lambda i: (0, 0)),
                  pl.BlockSpec((cin, c), lambda i: (0, 0)),
                  vec_c, vec_c, vec_c, vec_c],
        out_specs=pl.BlockSpec((tn, c), lambda i: (i, 0)),
        compiler_params=pltpu.CompilerParams(
            dimension_semantics=("parallel",)),
    )(conv, feat, a3, b3, w2, ws, a2, b2, a_s, b_s)


def kernel(features, points, neighbors, unary1_weight, kpconv_weights,
           kpconv_kernel_points, unary2_weight, unary_shortcut_weight):
    n = features.shape[0]
    tn_conv = 256

    ps, pq = _gemm_stats(features, unary1_weight, tn=4096)
    a1, b1 = _scale_shift(ps, pq, n)
    x1 = _gemm_affine(features, unary1_weight, a1, b1, tn=4096)

    kpt = kpconv_kernel_points
    pts_pad = jnp.concatenate(
        [points, jnp.full((1, 3), 1e6, jnp.float32),
         jnp.zeros((7, 3), jnp.float32)], axis=0)
    x1_pad = jnp.concatenate([x1, jnp.zeros((8, _CM), jnp.float32)], axis=0)
    skp = pts_pad @ kpt.T
    s2 = jnp.sum(pts_pad * pts_pad, axis=1, keepdims=True)
    src = jnp.concatenate(
        [x1_pad, skp, s2, pts_pad,
         jnp.zeros((n + 8, 128 - _CM - _KP - 1 - 3), jnp.float32)],
        axis=1).reshape(n + 8, 1, 128)

    kp2 = jnp.sum(kpt * kpt, axis=1)
    q2 = jnp.sum(points * points, axis=1, keepdims=True)
    t2 = 2.0 * (points @ kpt.T) + kp2[None, :] + q2
    t2 = jnp.concatenate([t2, jnp.full((n, 1), 1e30, jnp.float32)], axis=1)
    qaug = jnp.concatenate(
        [t2, points, jnp.zeros((n, 5), jnp.float32)],
        axis=1).reshape(n, 1, 24)

    idx3 = neighbors.astype(jnp.int32).reshape(n // tn_conv, 1, tn_conv * _H)

    nb_feat, w_out, inv3 = _kpconv(src, qaug, idx3, n, tn_conv)
    nbf = nb_feat.reshape(n, _H, _CM)
    wmat = w_out.reshape(n, _H, _KPAD)
    weighted = jnp.einsum('nhk,nhc->nkc', wmat, nbf).reshape(n, _KPAD * _CM)
    inv = inv3[:, 0, 0:1]

    w_pad = jnp.concatenate(
        [kpconv_weights.reshape(_KP * _CM, _CM),
         jnp.zeros((_CM, _CM), jnp.float32)], axis=0)

    conv, ps3, pq3 = _conv_gemm(weighted, w_pad, inv, tn=1024)
    a3, b3 = _scale_shift(ps3, pq3, n)

    ps2, pq2, pss, pqs = _final_stats(conv, features, a3, b3, unary2_weight,
                                      unary_shortcut_weight, tn=4096)
    a2, b2 = _scale_shift(ps2, pq2, n)
    a_s, b_s = _scale_shift(pss, pqs, n)
    return _final_apply(conv, features, a3, b3, unary2_weight,
                        unary_shortcut_weight, a2, b2, a_s, b_s, tn=2048)


# speedup vs baseline: 2.8061x; 2.8061x over previous
import functools

import jax
import jax.numpy as jnp
from jax import lax
from jax.experimental import pallas as pl
from jax.experimental.pallas import tpu as pltpu

_EPS = 1e-5
_SLOPE = 0.1
_KP_EXTENT = 1.2
_H = 16
_KP = 15
_KPAD = 16
_CM = 64


def _lrelu(y):
    return jnp.where(y >= 0.0, y, _SLOPE * y)


def _kpconv_kernel(idx_ref, src_ref, out_ref, ism_ref, sem):
    cp = pltpu.make_async_copy(idx_ref.at[0, 0], ism_ref, sem)
    cp.start()
    cp.wait()
    nrows = out_ref.shape[0]

    def body(r0, carry):
        for u in range(_H):
            r = r0 * _H + u
            out_ref[r] = src_ref[ism_ref[r]]
        return carry

    lax.fori_loop(0, nrows // _H, body, 0)


def _kpconv(src, idx3, n, tn):
    t = n // tn
    return pl.pallas_call(
        _kpconv_kernel,
        out_shape=jax.ShapeDtypeStruct((n * _H, 1, 128), jnp.float32),
        grid=(t,),
        in_specs=[pl.BlockSpec((1, 1, tn * _H), lambda i: (i, 0, 0)),
                  pl.BlockSpec(src.shape, lambda i: (0, 0, 0))],
        out_specs=pl.BlockSpec((tn * _H, 1, 128), lambda i: (i, 0, 0)),
        scratch_shapes=[pltpu.SMEM((tn * _H,), jnp.int32),
                        pltpu.SemaphoreType.DMA],
        compiler_params=pltpu.CompilerParams(
            dimension_semantics=("parallel",),
            disable_bounds_checks=True,
            vmem_limit_bytes=60 * 1024 * 1024),
    )(idx3, src)


def _gemm_stats_kernel(x_ref, w_ref, ps_ref, pq_ref):
    y = jnp.dot(x_ref[...], w_ref[...], preferred_element_type=jnp.float32)
    ps_ref[0, 0, :] = jnp.sum(y, axis=0)
    pq_ref[0, 0, :] = jnp.sum(y * y, axis=0)


def _gemm_stats(x, w, tn):
    n, kdim = x.shape
    c = w.shape[1]
    t = n // tn
    return pl.pallas_call(
        _gemm_stats_kernel,
        out_shape=(jax.ShapeDtypeStruct((t, 1, c), jnp.float32),
                   jax.ShapeDtypeStruct((t, 1, c), jnp.float32)),
        grid=(t,),
        in_specs=[pl.BlockSpec((tn, kdim), lambda i: (i, 0)),
                  pl.BlockSpec((kdim, c), lambda i: (0, 0))],
        out_specs=(pl.BlockSpec((1, 1, c), lambda i: (i, 0, 0)),
                   pl.BlockSpec((1, 1, c), lambda i: (i, 0, 0))),
        compiler_params=pltpu.CompilerParams(
            dimension_semantics=("parallel",)),
    )(x, w)


def _scale_shift(ps, pq, n):
    s = jnp.sum(ps[:, 0, :], axis=0)
    q = jnp.sum(pq[:, 0, :], axis=0)
    mean = s / n
    var = q / n - mean * mean
    inv = lax.rsqrt(var + _EPS)
    return inv[None, :], (-mean * inv)[None, :]


def _gemm_affine_kernel(x_ref, w_ref, a_ref, b_ref, o_ref):
    y = jnp.dot(x_ref[...], w_ref[...], preferred_element_type=jnp.float32)
    o_ref[...] = _lrelu(y * a_ref[...] + b_ref[...])


def _gemm_affine(x, w, a, b, tn):
    n, kdim = x.shape
    c = w.shape[1]
    t = n // tn
    return pl.pallas_call(
        _gemm_affine_kernel,
        out_shape=jax.ShapeDtypeStruct((n, c), jnp.float32),
        grid=(t,),
        in_specs=[pl.BlockSpec((tn, kdim), lambda i: (i, 0)),
                  pl.BlockSpec((kdim, c), lambda i: (0, 0)),
                  pl.BlockSpec((1, c), lambda i: (0, 0)),
                  pl.BlockSpec((1, c), lambda i: (0, 0))],
        out_specs=pl.BlockSpec((tn, c), lambda i: (i, 0)),
        compiler_params=pltpu.CompilerParams(
            dimension_semantics=("parallel",)),
    )(x, w, a, b)


def _conv_gemm_kernel(wt_ref, w_ref, inv_ref, o_ref, ps_ref, pq_ref):
    y = jnp.dot(wt_ref[...], w_ref[...], preferred_element_type=jnp.float32)
    y = y * inv_ref[...]
    o_ref[...] = y
    ps_ref[0, 0, :] = jnp.sum(y, axis=0)
    pq_ref[0, 0, :] = jnp.sum(y * y, axis=0)


def _conv_gemm(weighted, w, inv, tn):
    n, kdim = weighted.shape
    c = w.shape[1]
    t = n // tn
    return pl.pallas_call(
        _conv_gemm_kernel,
        out_shape=(jax.ShapeDtypeStruct((n, c), jnp.float32),
                   jax.ShapeDtypeStruct((t, 1, c), jnp.float32),
                   jax.ShapeDtypeStruct((t, 1, c), jnp.float32)),
        grid=(t,),
        in_specs=[pl.BlockSpec((tn, kdim), lambda i: (i, 0)),
                  pl.BlockSpec((kdim, c), lambda i: (0, 0)),
                  pl.BlockSpec((tn, 1), lambda i: (i, 0))],
        out_specs=(pl.BlockSpec((tn, c), lambda i: (i, 0)),
                   pl.BlockSpec((1, 1, c), lambda i: (i, 0, 0)),
                   pl.BlockSpec((1, 1, c), lambda i: (i, 0, 0))),
        compiler_params=pltpu.CompilerParams(
            dimension_semantics=("parallel",)),
    )(weighted, w, inv)


def _final_stats_kernel(cv_ref, ft_ref, a3_ref, b3_ref, w2_ref, ws_ref,
                        ps2_ref, pq2_ref, pss_ref, pqs_ref):
    x3 = _lrelu(cv_ref[...] * a3_ref[...] + b3_ref[...])
    y2 = jnp.dot(x3, w2_ref[...], preferred_element_type=jnp.float32)
    sc = jnp.dot(ft_ref[...], ws_ref[...], preferred_element_type=jnp.float32)
    ps2_ref[0, 0, :] = jnp.sum(y2, axis=0)
    pq2_ref[0, 0, :] = jnp.sum(y2 * y2, axis=0)
    pss_ref[0, 0, :] = jnp.sum(sc, axis=0)
    pqs_ref[0, 0, :] = jnp.sum(sc * sc, axis=0)


def _final_stats(conv, feat, a3, b3, w2, ws, tn):
    n, cm = conv.shape
    cin = feat.shape[1]
    c = w2.shape[1]
    t = n // tn
    stat = jax.ShapeDtypeStruct((t, 1, c), jnp.float32)
    return pl.pallas_call(
        _final_stats_kernel,
        out_shape=(stat, stat, stat, stat),
        grid=(t,),
        in_specs=[pl.BlockSpec((tn, cm), lambda i: (i, 0)),
                  pl.BlockSpec((tn, cin), lambda i: (i, 0)),
                  pl.BlockSpec((1, cm), lambda i: (0, 0)),
                  pl.BlockSpec((1, cm), lambda i: (0, 0)),
                  pl.BlockSpec((cm, c), lambda i: (0, 0)),
                  pl.BlockSpec((cin, c), lambda i: (0, 0))],
        out_specs=(pl.BlockSpec((1, 1, c), lambda i: (i, 0, 0)),) * 4,
        compiler_params=pltpu.CompilerParams(
            dimension_semantics=("parallel",)),
    )(conv, feat, a3, b3, w2, ws)


def _final_apply_kernel(cv_ref, ft_ref, a3_ref, b3_ref, w2_ref, ws_ref,
                        a2_ref, b2_ref, as_ref, bs_ref, o_ref):
    x3 = _lrelu(cv_ref[...] * a3_ref[...] + b3_ref[...])
    y2 = jnp.dot(x3, w2_ref[...], preferred_element_type=jnp.float32)
    sc = jnp.dot(ft_ref[...], ws_ref[...], preferred_element_type=jnp.float32)
    y = y2 * a2_ref[...] + b2_ref[...] + (sc * as_ref[...] + bs_ref[...])
    o_ref[...] = _lrelu(y)


def _final_apply(conv, feat, a3, b3, w2, ws, a2, b2, a_s, b_s, tn):
    n, cm = conv.shape
    cin = feat.shape[1]
    c = w2.shape[1]
    t = n // tn
    vec_c = pl.BlockSpec((1, c), lambda i: (0, 0))
    return pl.pallas_call(
        _final_apply_kernel,
        out_shape=jax.ShapeDtypeStruct((n, c), jnp.float32),
        grid=(t,),
        in_specs=[pl.BlockSpec((tn, cm), lambda i: (i, 0)),
                  pl.BlockSpec((tn, cin), lambda i: (i, 0)),
                  pl.BlockSpec((1, cm), lambda i: (0, 0)),
                  pl.BlockSpec((1, cm), lambda i: (0, 0)),
                  pl.BlockSpec((cm, c), lambda i: (0, 0)),
                  pl.BlockSpec((cin, c), lambda i: (0, 0)),
                  vec_c, vec_c, vec_c, vec_c],
        out_specs=pl.BlockSpec((tn, c), lambda i: (i, 0)),
        compiler_params=pltpu.CompilerParams(
            dimension_semantics=("parallel",)),
    )(conv, feat, a3, b3, w2, ws, a2, b2, a_s, b_s)


def kernel(features, points, neighbors, unary1_weight, kpconv_weights,
           kpconv_kernel_points, unary2_weight, unary_shortcut_weight):
    n = features.shape[0]
    tn_conv = 256

    ps, pq = _gemm_stats(features, unary1_weight, tn=4096)
    a1, b1 = _scale_shift(ps, pq, n)
    x1 = _gemm_affine(features, unary1_weight, a1, b1, tn=4096)

    kpt = kpconv_kernel_points
    pts_pad = jnp.concatenate(
        [points, jnp.full((1, 3), 1e6, jnp.float32),
         jnp.zeros((7, 3), jnp.float32)], axis=0)
    x1_pad = jnp.concatenate([x1, jnp.zeros((8, _CM), jnp.float32)], axis=0)
    skp = pts_pad @ kpt.T
    s2 = jnp.sum(pts_pad * pts_pad, axis=1, keepdims=True)
    src = jnp.concatenate(
        [x1_pad, skp, s2, pts_pad,
         jnp.zeros((n + 8, 128 - _CM - _KP - 1 - 3), jnp.float32)],
        axis=1).reshape(n + 8, 1, 128)

    idx3 = neighbors.astype(jnp.int32).reshape(n // tn_conv, 1, tn_conv * _H)
    rows = _kpconv(src, idx3, n, tn_conv).reshape(n, _H, 128)

    feat = rows[:, :, 0:_CM]
    skp = rows[:, :, _CM:_CM + _KP]
    s2v = rows[:, :, 79:80]
    sxyz = rows[:, :, 80:83]
    kp2 = jnp.sum(kpt * kpt, axis=1)
    q2 = jnp.sum(points * points, axis=1, keepdims=True)
    t2m = 2.0 * (points @ kpt.T) + kp2[None, :] + q2
    sdq = jnp.sum(sxyz * points[:, None, :], axis=2, keepdims=True)
    sqd = jnp.maximum(s2v - 2.0 * sdq - 2.0 * skp + t2m[:, None, :], 0.0)
    all_w = jnp.maximum(1.0 - jnp.sqrt(sqd) * (1.0 / _KP_EXTENT), 0.0)

    weighted = jnp.einsum('nhk,nhc->nkc', all_w, feat).reshape(n, _KP * _CM)
    nb_sum = jnp.sum(feat, axis=2)
    cnt = jnp.maximum(jnp.sum((nb_sum > 0.0).astype(jnp.int32), axis=1), 1)
    inv = (1.0 / cnt.astype(jnp.float32)).reshape(n, 1)

    conv, ps3, pq3 = _conv_gemm(
        weighted, kpconv_weights.reshape(_KP * _CM, _CM), inv, tn=1024)
    a3, b3 = _scale_shift(ps3, pq3, n)

    ps2, pq2, pss, pqs = _final_stats(conv, features, a3, b3, unary2_weight,
                                      unary_shortcut_weight, tn=4096)
    a2, b2 = _scale_shift(ps2, pq2, n)
    a_s, b_s = _scale_shift(pss, pqs, n)
    return _final_apply(conv, features, a3, b3, unary2_weight,
                        unary_shortcut_weight, a2, b2, a_s, b_s, tn=2048)
